# baseline (speedup 1.0000x reference)
import functools

import jax
import jax.numpy as jnp
from jax import lax
from jax.experimental import pallas as pl
from jax.experimental.pallas import tpu as pltpu
from jax.experimental.pallas import tpu_sc as plsc

B = 8192
D = 1024
H = 1024
E = 8
ND = 3
TOPK = 2

A = B * TOPK
TM = 256
NT = A // TM + E
L = NT * TM
NTP = 80

BT1 = 512


def _gate_body(x_ref, lab_ref, emb_ref, gw_ref, gb_ref,
               topi_ref, probs_ref):
    gw = gw_ref[...]
    gwx = gw[:, :D].astype(jnp.bfloat16)
    gwd = gw[:, D:].astype(jnp.bfloat16)
    de_log = lax.dot_general(emb_ref[...].astype(jnp.bfloat16), gwd,
                             (((1,), (1,)), ((), ())),
                             preferred_element_type=jnp.float32)
    lab = lab_ref[...]
    de = jnp.where(lab == 0, de_log[0][None, :],
                   jnp.where(lab == 1, de_log[1][None, :],
                             de_log[2][None, :]))
    logits = (lax.dot_general(x_ref[...].astype(jnp.bfloat16), gwx,
                              (((1,), (1,)), ((), ())),
                              preferred_element_type=jnp.float32)
              + de + gb_ref[...])
    eiota = lax.broadcasted_iota(jnp.int32, (BT1, E), 1)
    v1 = jnp.max(logits, axis=1, keepdims=True)
    i1 = jnp.min(jnp.where(logits == v1, eiota, E), axis=1, keepdims=True)
    masked = jnp.where(eiota == i1, -jnp.inf, logits)
    v2 = jnp.max(masked, axis=1, keepdims=True)
    i2 = jnp.min(jnp.where(masked == v2, eiota, E), axis=1, keepdims=True)
    e21 = jnp.exp(v2 - v1)
    p1 = 1.0 / (1.0 + e21)
    p2 = e21 / (1.0 + e21)
    topi_ref[...] = jnp.concatenate([i1, i2], axis=1)
    probs_ref[...] = jnp.concatenate([p1, p2], axis=1)


def _gate(x, lab, emb, gate_W, gb):
    return pl.pallas_call(
        _gate_body,
        grid=(B // BT1,),
        in_specs=[
            pl.BlockSpec((BT1, D), lambda t: (t, 0)),
            pl.BlockSpec((BT1, 1), lambda t: (t, 0)),
            pl.BlockSpec((ND, D), lambda t: (0, 0)),
            pl.BlockSpec((E, 2 * D), lambda t: (0, 0)),
            pl.BlockSpec((1, E), lambda t: (0, 0)),
        ],
        out_specs=[
            pl.BlockSpec((BT1, TOPK), lambda t: (t, 0)),
            pl.BlockSpec((BT1, TOPK), lambda t: (t, 0)),
        ],
        out_shape=[
            jax.ShapeDtypeStruct((B, TOPK), jnp.int32),
            jax.ShapeDtypeStruct((B, TOPK), jnp.float32),
        ],
    )(x, lab, emb, gate_W, gb)


NSUB = 16
NA_SUB = A // NSUB
NV = NA_SUB // 16


NWC = 32
NA_CW = A // NWC


def _count_body(topi_hbm, cnt_hbm, eid_v, stage16, sem):
    c = lax.axis_index("c")
    s = lax.axis_index("s")
    wid = s * 2 + c
    lanes = lax.broadcasted_iota(jnp.int32, (16,), 0)
    zero16 = jnp.zeros((16,), jnp.int32)
    a0 = wid * NA_CW
    pltpu.sync_copy(topi_hbm.at[pl.ds(a0, NA_CW)], eid_v)

    def count_body(i, cnt):
        v = eid_v[pl.ds(i * 16, 16)]
        for e in range(E):
            m = v == e
            pc = plsc.all_reduce_population_count(m)
            cnt = cnt + jnp.where(lanes == e, pc, zero16)
        return cnt

    cnt = lax.fori_loop(0, NA_CW // 16, count_body, zero16)
    stage16[...] = cnt
    pltpu.sync_copy(stage16, cnt_hbm.at[wid])


def _pos_body(topi_hbm, cnt_hbm, pos_hbm, et_hbm,
              eid_v, posbuf, cntall, basebuf, etbuf, sem):
    c = lax.axis_index("c")
    s = lax.axis_index("s")
    wid = s * 2 + c
    lanes = lax.broadcasted_iota(jnp.int32, (16,), 0)
    zero16 = jnp.zeros((16,), jnp.int32)
    pltpu.sync_copy(cnt_hbm, cntall)
    total = zero16
    prefix = zero16
    for w in range(NWC):
        row = cntall[w]
        total = total + row
        if w > 0:
            prefix = prefix + jnp.where((zero16 + w) <= (zero16 + wid),
                                        cntall[w - 1], zero16)
    pc_ = ((total + (TM - 1)) >> 8) << 8
    po = plsc.cumsum(pc_) - pc_
    basebuf[...] = po + prefix

    a0 = wid * NA_CW
    pltpu.sync_copy(topi_hbm.at[pl.ds(a0, NA_CW)], eid_v)

    def pos_body(i, _):
        v = eid_v[pl.ds(i * 16, 16)]
        b = plsc.load_gather(basebuf, [v])
        rank = zero16
        delta = zero16
        for e in range(E):
            m = v == e
            cs = plsc.cumsum(m.astype(jnp.int32))
            rank = rank + jnp.where(m, cs - 1, zero16)
            pcn = plsc.all_reduce_population_count(m)
            delta = delta + jnp.where(lanes == e, pcn, zero16)
        posbuf[pl.ds(i * 16, 16)] = b + rank
        basebuf[...] = basebuf[...] + delta
        return 0

    lax.fori_loop(0, NA_CW // 16, pos_body, 0)
    pltpu.sync_copy(posbuf, pos_hbm.at[pl.ds(a0, NA_CW)])

    @pl.when(wid == 0)
    def _():
        for j in range(NTP // 16):
            start = (lanes + j * 16) * TM
            acc = zero16
            for e in range(E):
                poe = zero16 + po[e]
                pce = zero16 + pc_[e]
                sel = (start >= poe) & (start < poe + pce)
                acc = acc + jnp.where(sel, jnp.full((16,), e, jnp.int32),
                                      zero16)
            etbuf[pl.ds(j * 16, 16)] = acc
        pltpu.sync_copy(etbuf, et_hbm)


def _route(topi_flat):
    mesh = plsc.VectorSubcoreMesh(core_axis_name="c", subcore_axis_name="s")
    cp = pltpu.CompilerParams(needs_layout_passes=False)
    count = pl.kernel(
        _count_body,
        out_type=jax.ShapeDtypeStruct((NWC, 16), jnp.int32),
        mesh=mesh,
        compiler_params=cp,
        scratch_types=[
            pltpu.VMEM((NA_CW,), jnp.int32),
            pltpu.VMEM((16,), jnp.int32),
            pltpu.SemaphoreType.DMA,
        ],
    )
    cnts = count(topi_flat)
    posk = pl.kernel(
        _pos_body,
        out_type=[
            jax.ShapeDtypeStruct((A,), jnp.int32),
            jax.ShapeDtypeStruct((NTP,), jnp.int32),
        ],
        mesh=mesh,
        compiler_params=cp,
        scratch_types=[
            pltpu.VMEM((NA_CW,), jnp.int32),
            pltpu.VMEM((NA_CW,), jnp.int32),
            pltpu.VMEM((NWC, 16), jnp.int32),
            pltpu.VMEM((16,), jnp.int32),
            pltpu.VMEM((NTP,), jnp.int32),
            pltpu.SemaphoreType.DMA,
        ],
    )
    return posk(topi_flat, cnts)


NW = 32
NTOK_W = B // NW
CH = 16


def _disp_body(pos_hbm, pr_hbm, x_hbm, xd_hbm, wd_hbm,
               pos_v, pr_v, xbuf, wbuf, idxb, sem, sem2, sem3, sem4):
    c = lax.axis_index("c")
    s = lax.axis_index("s")
    wid = s * 2 + c
    lanes = lax.broadcasted_iota(jnp.int32, (16,), 0)
    tok0 = wid * NTOK_W
    a0 = wid * NTOK_W * 2
    pltpu.sync_copy(pos_hbm.at[pl.ds(a0, NTOK_W * 2)], pos_v)
    pltpu.sync_copy(pr_hbm.at[pl.ds(a0, NTOK_W * 2)], pr_v)
    for j in range(NTOK_W // CH):
        pltpu.sync_copy(x_hbm.at[pl.ds(tok0 + j * CH, CH)], xbuf)
        idxb[2 * j, :] = plsc.load_gather(pos_v, [j * 2 * CH + 2 * lanes])
        idxb[2 * j + 1, :] = plsc.load_gather(pos_v,
                                              [j * 2 * CH + 2 * lanes + 1])
        wbuf[0, :] = plsc.load_gather(pr_v, [j * 2 * CH + 2 * lanes])
        wbuf[1, :] = plsc.load_gather(pr_v, [j * 2 * CH + 2 * lanes + 1])
        cp1 = pltpu.async_copy(xbuf, xd_hbm.at[idxb.at[2 * j]], sem)
        cp2 = pltpu.async_copy(xbuf, xd_hbm.at[idxb.at[2 * j + 1]], sem2)
        cp3 = pltpu.async_copy(wbuf.at[0], wd_hbm.at[idxb.at[2 * j]], sem3)
        cp4 = pltpu.async_copy(wbuf.at[1], wd_hbm.at[idxb.at[2 * j + 1]],
                               sem4)
        cp1.wait()
        cp2.wait()
        cp3.wait()
        cp4.wait()


def _dispatch(pos, probs_flat, x):
    mesh = plsc.VectorSubcoreMesh(core_axis_name="c", subcore_axis_name="s")
    f = pl.kernel(
        _disp_body,
        out_type=[
            jax.ShapeDtypeStruct((L, D), jnp.float32),
            jax.ShapeDtypeStruct((L,), jnp.float32),
        ],
        mesh=mesh,
        compiler_params=pltpu.CompilerParams(needs_layout_passes=False),
        scratch_types=[
            pltpu.VMEM((NTOK_W * 2,), jnp.int32),
            pltpu.VMEM((NTOK_W * 2,), jnp.float32),
            pltpu.VMEM((CH, D), jnp.float32),
            pltpu.VMEM((2, 16), jnp.float32),
            pltpu.VMEM((2 * NTOK_W // CH, 16), jnp.int32),
            pltpu.SemaphoreType.DMA,
            pltpu.SemaphoreType.DMA,
            pltpu.SemaphoreType.DMA,
            pltpu.SemaphoreType.DMA,
        ],
    )
    return f(pos, probs_flat, x)


def _mm_body(et_ref, xd_ref, w_ref, b_ref, wd_ref, y_ref):
    xb = xd_ref[...].astype(jnp.bfloat16)
    wb = w_ref[0].astype(jnp.bfloat16)
    mm = lax.dot_general(xb, wb, (((1,), (1,)), ((), ())),
                         preferred_element_type=jnp.float32)
    wcol = wd_ref[...].reshape(TM, 1)
    y_ref[...] = wcol * (mm + b_ref[0])


def _expert_mm(et, xd, W, b, wd3):
    grid_spec = pltpu.PrefetchScalarGridSpec(
        num_scalar_prefetch=1,
        grid=(NT,),
        in_specs=[
            pl.BlockSpec((TM, D), lambda t, et: (t, 0)),
            pl.BlockSpec((1, H, D), lambda t, et: (et[t], 0, 0)),
            pl.BlockSpec((1, 1, H), lambda t, et: (et[t], 0, 0)),
            pl.BlockSpec((1, TM, 1), lambda t, et: (t, 0, 0)),
        ],
        out_specs=pl.BlockSpec((TM, H), lambda t, et: (t, 0)),
    )
    return pl.pallas_call(
        _mm_body,
        grid_spec=grid_spec,
        out_shape=jax.ShapeDtypeStruct((L, H), jnp.float32),
    )(et, xd, W, b, wd3)


def _comb_body(y_hbm, pos_hbm, out_hbm, idx_v, rows, obuf, sem, sem2):
    c = lax.axis_index("c")
    s = lax.axis_index("s")
    wid = s * 2 + c
    tok0 = wid * NTOK_W

    def chunk(j, _):
        pltpu.sync_copy(pos_hbm.at[pl.ds(wid * NTOK_W * 2 + j * 2 * CH,
                                         2 * CH)], idx_v)
        pltpu.async_copy(y_hbm.at[idx_v], rows, sem).wait()

        def col(cc, _):
            for i in range(CH):
                obuf[i, pl.ds(cc * 16, 16)] = (
                    rows[2 * i, pl.ds(cc * 16, 16)]
                    + rows[2 * i + 1, pl.ds(cc * 16, 16)])
            return 0

        lax.fori_loop(0, H // 16, col, 0)
        pltpu.sync_copy(obuf, out_hbm.at[pl.ds(tok0 + j * CH, CH)])
        return 0

    lax.fori_loop(0, NTOK_W // CH, chunk, 0)


def _combine(y, pos):
    mesh = plsc.VectorSubcoreMesh(core_axis_name="c", subcore_axis_name="s")
    f = pl.kernel(
        _comb_body,
        out_type=jax.ShapeDtypeStruct((B, H), jnp.float32),
        mesh=mesh,
        compiler_params=pltpu.CompilerParams(needs_layout_passes=False),
        scratch_types=[
            pltpu.VMEM((2 * CH,), jnp.int32),
            pltpu.VMEM((2 * CH, H), jnp.float32),
            pltpu.VMEM((CH, H), jnp.float32),
            pltpu.SemaphoreType.DMA,
            pltpu.SemaphoreType.DMA,
        ],
    )
    return f(y, pos)


_STAGE = 5


def kernel(x, difficulty_labels, W_experts, b_experts, emb, gate_W, gate_b):
    lab = difficulty_labels.astype(jnp.int32).reshape(B, 1)
    gb = gate_b.reshape(1, E)
    topi, probs = _gate(x, lab, emb, gate_W, gb)
    if _STAGE == 1:
        return (topi, probs)
    pos, et = _route(topi.reshape(A))
    if _STAGE == 2:
        return (pos, et, topi)
    xd, wd = _dispatch(pos, probs.reshape(A), x)
    if _STAGE == 3:
        return (xd, wd, topi)
    y = _expert_mm(et[:NT], xd, W_experts, b_experts.reshape(E, 1, H),
                   wd.reshape(NT, TM, 1))
    if _STAGE == 4:
        return (y, topi)
    out = _combine(y, pos)
    return (out, topi)
